# baseline (speedup 1.0000x reference)
import jax
import jax.numpy as jnp
from jax.experimental import pallas as pl
from jax.experimental.pallas import tpu as pltpu

D_MODEL_K = 1024
N_EXPERTS_K = 16
TOP_K_K = 8


def _moe_body(h_ref, gw_ref, ew_ref, out_ref, logits_ref, comb_ref):
    e = pl.program_id(0)
    col = jax.lax.broadcasted_iota(jnp.int32, (h_ref.shape[0], N_EXPERTS_K), 1)

    @pl.when(e == 0)
    def _router():
        h = h_ref[...]
        logits = jax.lax.dot_general(
            h, gw_ref[...], (((1,), (1,)), ((), ())),
            preferred_element_type=jnp.float32)
        logits_ref[...] = logits
        m = jnp.max(logits, axis=1, keepdims=True)
        ex = jnp.exp(logits - m)
        w = ex / jnp.sum(ex, axis=1, keepdims=True)
        rank = jnp.zeros(logits.shape, jnp.int32)
        for j in range(N_EXPERTS_K):
            lj = logits[:, j:j + 1]
            rank += ((lj > logits)
                     | ((lj == logits) & (j < col))).astype(jnp.int32)
        comb_ref[...] = jnp.where(rank < TOP_K_K, w, 0.0)

    y = jax.lax.dot_general(
        h_ref[...], ew_ref[0], (((1,), (1,)), ((), ())),
        preferred_element_type=jnp.float32)
    c = jnp.sum(jnp.where(col == e, comb_ref[...], 0.0), axis=1, keepdims=True)
    contrib = c * y

    @pl.when(e == 0)
    def _init():
        out_ref[...] = contrib

    @pl.when(e > 0)
    def _acc():
        out_ref[...] += contrib


@jax.jit
def kernel(hidden_states, gate_w, expert_w):
    b, t, d = hidden_states.shape
    h_flat = hidden_states.reshape(t * b, d)
    n_exp = expert_w.shape[0]

    out, logits = pl.pallas_call(
        _moe_body,
        grid=(n_exp,),
        in_specs=[
            pl.BlockSpec((t * b, d), lambda e: (0, 0)),
            pl.BlockSpec((n_exp, d), lambda e: (0, 0)),
            pl.BlockSpec((1, d, d), lambda e: (e, 0, 0)),
        ],
        out_specs=[
            pl.BlockSpec((t * b, d), lambda e: (0, 0)),
            pl.BlockSpec((t * b, n_exp), lambda e: (0, 0)),
        ],
        out_shape=[
            jax.ShapeDtypeStruct((t * b, d), jnp.float32),
            jax.ShapeDtypeStruct((t * b, n_exp), jnp.float32),
        ],
        scratch_shapes=[pltpu.VMEM((t * b, n_exp), jnp.float32)],
    )(h_flat, gate_w, expert_w)
    return out.reshape(b, t, d), logits


# speedup vs baseline: 1.0219x; 1.0219x over previous
import jax
import jax.numpy as jnp
from jax.experimental import pallas as pl
from jax.experimental.pallas import tpu as pltpu

D_MODEL_K = 1024
N_EXPERTS_K = 16
TOP_K_K = 8


def _moe_body(h_ref, gw_ref, ew_ref, out_ref, logits_ref, comb_ref):
    e = pl.program_id(0)
    col = jax.lax.broadcasted_iota(jnp.int32, (h_ref.shape[0], N_EXPERTS_K), 1)

    @pl.when(e == 0)
    def _router():
        h = h_ref[...]
        logits = jax.lax.dot_general(
            h, gw_ref[...], (((1,), (1,)), ((), ())),
            preferred_element_type=jnp.float32)
        logits_ref[...] = logits
        m = jnp.max(logits, axis=1, keepdims=True)
        ex = jnp.exp(logits - m)
        w = ex / jnp.sum(ex, axis=1, keepdims=True)
        rank = jnp.zeros(logits.shape, jnp.int32)
        for j in range(N_EXPERTS_K):
            lj = logits[:, j:j + 1]
            rank += (lj > logits).astype(jnp.int32)
            rank += ((lj == logits) & (j < col)).astype(jnp.int32)
        comb_ref[...] = jnp.where(rank < TOP_K_K, w, 0.0)

    y = jax.lax.dot_general(
        h_ref[...], ew_ref[0], (((1,), (1,)), ((), ())),
        preferred_element_type=jnp.float32)
    c = jnp.sum(jnp.where(col == e, comb_ref[...], 0.0), axis=1, keepdims=True)
    contrib = c * y

    @pl.when(e == 0)
    def _init():
        out_ref[...] = contrib

    @pl.when(e > 0)
    def _acc():
        out_ref[...] += contrib


@jax.jit
def kernel(hidden_states, gate_w, expert_w):
    b, t, d = hidden_states.shape
    h_flat = hidden_states.reshape(t * b, d)
    n_exp = expert_w.shape[0]

    out, logits = pl.pallas_call(
        _moe_body,
        grid=(n_exp,),
        in_specs=[
            pl.BlockSpec((t * b, d), lambda e: (0, 0)),
            pl.BlockSpec((n_exp, d), lambda e: (0, 0)),
            pl.BlockSpec((1, d, d), lambda e: (e, 0, 0)),
        ],
        out_specs=[
            pl.BlockSpec((t * b, d), lambda e: (0, 0)),
            pl.BlockSpec((t * b, n_exp), lambda e: (0, 0)),
        ],
        out_shape=[
            jax.ShapeDtypeStruct((t * b, d), jnp.float32),
            jax.ShapeDtypeStruct((t * b, n_exp), jnp.float32),
        ],
        scratch_shapes=[pltpu.VMEM((t * b, n_exp), jnp.float32)],
    )(h_flat, gate_w, expert_w)
    return out.reshape(b, t, d), logits
